# Initial kernel scaffold; baseline (speedup 1.0000x reference)
import jax
import jax.numpy as jnp
from jax.experimental import pallas as pl


def kernel(x, mask, table):
    raise NotImplementedError("write your pallas kernel here")


# speedup vs baseline: 40.2006x; 40.2006x over previous
import functools

import jax
import jax.numpy as jnp
from jax import lax
from jax.experimental import pallas as pl
from jax.experimental.pallas import tpu as pltpu
from jax.experimental.pallas import tpu_sc as plsc

B, S, V, D = 4096, 200, 50, 32
L = 16
NW = 32
ROWS = B // NW
SP = 208
VP = 64
NCHUNK = SP // L


def _sc_hist(xp, mf, zeros):
    mesh = plsc.VectorSubcoreMesh(core_axis_name="c", subcore_axis_name="s")

    @functools.partial(
        pl.kernel,
        mesh=mesh,
        compiler_params=pltpu.CompilerParams(needs_layout_passes=False),
        out_type=jax.ShapeDtypeStruct((B * VP,), jnp.float32),
        scratch_types=[
            pltpu.VMEM((ROWS * SP,), jnp.int32),
            pltpu.VMEM((ROWS * SP,), jnp.float32),
            pltpu.VMEM((ROWS * VP,), jnp.float32),
        ],
    )
    def hist(x_hbm, m_hbm, z_hbm, cnt_hbm, x_v, m_v, cnt_v):
        wid = lax.axis_index("s") * 2 + lax.axis_index("c")
        base = wid * ROWS
        pltpu.sync_copy(x_hbm.at[pl.ds(base * SP, ROWS * SP)], x_v)
        pltpu.sync_copy(m_hbm.at[pl.ds(base * SP, ROWS * SP)], m_v)
        pltpu.sync_copy(z_hbm, cnt_v)

        def row(r, carry):
            roff = jnp.full((L,), r * VP, jnp.int32)
            for k in range(NCHUNK):
                idx = x_v[pl.ds(r * SP + k * L, L)] + roff
                val = m_v[pl.ds(r * SP + k * L, L)]
                plsc.addupdate_scatter(cnt_v, [idx], val)
            return carry

        lax.fori_loop(0, ROWS, row, 0)
        pltpu.sync_copy(cnt_v, cnt_hbm.at[pl.ds(base * VP, ROWS * VP)])

    return hist(xp, mf, zeros)


def _tc_matmul(counts, tablep):
    blk = 512

    def body(c_ref, t_ref, o_ref):
        o_ref[...] = jnp.dot(c_ref[...], t_ref[...],
                             preferred_element_type=jnp.float32,
                             precision=jax.lax.Precision.HIGHEST)

    return pl.pallas_call(
        body,
        grid=(B // blk,),
        in_specs=[
            pl.BlockSpec((blk, VP), lambda i: (i, 0)),
            pl.BlockSpec((VP, D), lambda i: (0, 0)),
        ],
        out_specs=pl.BlockSpec((blk, D), lambda i: (i, 0)),
        out_shape=jax.ShapeDtypeStruct((B, D), jnp.float32),
    )(counts, tablep)


def kernel(x, mask, table):
    xp = jnp.pad(x, ((0, 0), (0, SP - S))).reshape(B * SP)
    mf = jnp.pad(mask.astype(jnp.float32), ((0, 0), (0, SP - S))).reshape(B * SP)
    zeros = jnp.zeros((ROWS * VP,), jnp.float32)
    tablep = jnp.pad(table, ((0, VP - V), (0, 0)))
    counts = _sc_hist(xp, mf, zeros).reshape(B, VP)
    return _tc_matmul(counts, tablep)
